# baseline (speedup 1.0000x reference)
import functools

import jax
import jax.numpy as jnp
from jax import lax
from jax.experimental import pallas as pl
from jax.experimental.pallas import tpu as pltpu
from jax.experimental.pallas import tpu_sc as plsc

HID = 128
N_NODES = 10000
N_EDGES = 320000
EPS = 1e-5

NC, NS, LANES = 2, 16, 16
NW = NC * NS
EPW = N_EDGES // NW
CHUNK = 80
NCHUNK = EPW // CHUNK
NPAD = 10240
ROWS_PER_TILE = NPAD // NS

def _sc_gather_body(node_hbm, src_hbm, dst_hbm, g_hbm,
                    idx_s, idx_d, ba0, bb0, ba1, bb1,
                    sa0, sb0, sa1, sb1, ss0, ss1):
    wid = lax.axis_index("s") * NC + lax.axis_index("c")
    base = wid * EPW

    pltpu.sync_copy(src_hbm.at[wid], idx_s)
    pltpu.sync_copy(dst_hbm.at[wid], idx_d)

    def issue(ci, ba, bb, sa, sb):
        pltpu.async_copy(node_hbm.at[idx_s.at[ci]], ba, sa)
        pltpu.async_copy(node_hbm.at[idx_d.at[ci]], bb, sb)

    def drain(ci, ba, bb, sa, sb, ss):
        pltpu.make_async_copy(node_hbm.at[idx_s.at[ci]], ba, sa).wait()
        pltpu.make_async_copy(node_hbm.at[idx_d.at[ci]], bb, sb).wait()

        def row(r, c2):
            for j in range(HID // LANES):
                sl = pl.ds(j * LANES, LANES)
                plsc.addupdate(ba.at[r, sl], bb[r, sl])
            return c2

        lax.fori_loop(0, CHUNK, row, 0)
        pltpu.async_copy(ba, g_hbm.at[pl.ds(base + ci * CHUNK, CHUNK)], ss)

    def wait_store(ci, ba, ss):
        pltpu.make_async_copy(ba, g_hbm.at[pl.ds(base + ci * CHUNK, CHUNK)],
                              ss).wait()

    issue(0, ba0, bb0, sa0, sb0)

    def step(c, carry):
        nxt = c + 1

        @pl.when(jnp.logical_and(nxt < NCHUNK, nxt % 2 == 0))
        def _():
            @pl.when(nxt >= 2)
            def _():
                wait_store(nxt - 2, ba0, ss0)
            issue(nxt, ba0, bb0, sa0, sb0)

        @pl.when(jnp.logical_and(nxt < NCHUNK, nxt % 2 == 1))
        def _():
            @pl.when(nxt >= 2)
            def _():
                wait_store(nxt - 2, ba1, ss1)
            issue(nxt, ba1, bb1, sa1, sb1)

        @pl.when(c % 2 == 0)
        def _():
            drain(c, ba0, bb0, sa0, sb0, ss0)

        @pl.when(c % 2 == 1)
        def _():
            drain(c, ba1, bb1, sa1, sb1, ss1)

        return carry

    lax.fori_loop(0, NCHUNK, step, 0)
    wait_store(NCHUNK - 2, ba1 if (NCHUNK - 2) % 2 else ba0,
               ss1 if (NCHUNK - 2) % 2 else ss0)
    wait_store(NCHUNK - 1, ba1 if (NCHUNK - 1) % 2 else ba0,
               ss1 if (NCHUNK - 1) % 2 else ss0)


def _sc_scatter_body(h2_hbm, src_hbm, dst_hbm, scsh_hbm, acc_hbm, eout_hbm,
                     is0, is1, id0, id1, eb0, eb1, scsh, accum,
                     sl0, sl1, si0, si1, sd0, sd1, so0, so1):
    cid = lax.axis_index("c")
    sid = lax.axis_index("s")
    wid = sid * NC + cid
    base = wid * EPW
    nj = HID // LANES

    pltpu.sync_copy(scsh_hbm, scsh)

    def zrow(r, carry):
        for j in range(nj):
            eb0[r, pl.ds(j * LANES, LANES)] = jnp.zeros((LANES,), jnp.float32)
        return carry

    lax.fori_loop(0, CHUNK, zrow, 0)

    def zcp(k, carry):
        rows = pl.ds(sid * ROWS_PER_TILE + k * CHUNK, CHUNK)
        pltpu.sync_copy(eb0, accum.at[rows])
        return carry

    lax.fori_loop(0, ROWS_PER_TILE // CHUNK, zcp, 0)
    plsc.subcore_barrier()

    def issue(ci, eb, sl, isb, si, idb, sd):
        pltpu.async_copy(h2_hbm.at[pl.ds(base + ci * CHUNK, CHUNK)], eb, sl)
        pltpu.async_copy(src_hbm.at[wid, ci], isb, si)
        pltpu.async_copy(dst_hbm.at[wid, ci], idb, sd)

    def drain(ci, eb, sl, isb, si, idb, sd, so):
        pltpu.make_async_copy(
            h2_hbm.at[pl.ds(base + ci * CHUNK, CHUNK)], eb, sl).wait()
        pltpu.make_async_copy(src_hbm.at[wid, ci], isb, si).wait()
        pltpu.make_async_copy(dst_hbm.at[wid, ci], idb, sd).wait()
        scs = tuple(scsh[pl.ds(j * LANES, LANES)] for j in range(nj))
        shs = tuple(scsh[pl.ds(HID + j * LANES, LANES)] for j in range(nj))

        def row(r, carry):
            cs, ch = carry
            for j in range(nj):
                sl_ = pl.ds(j * LANES, LANES)
                eb[r, sl_] = jnp.maximum(eb[r, sl_] * cs[j] + ch[j], 0.0)
            return carry

        lax.fori_loop(0, CHUNK, row, (scs, shs))
        pltpu.async_copy(eb, eout_hbm.at[pl.ds(base + ci * CHUNK, CHUNK)], so)
        pltpu.sync_copy(eb, accum.at[isb], add=True)
        pltpu.sync_copy(eb, accum.at[idb], add=True)

    def wait_store(ci, eb, so):
        pltpu.make_async_copy(
            eb, eout_hbm.at[pl.ds(base + ci * CHUNK, CHUNK)], so).wait()

    issue(0, eb0, sl0, is0, si0, id0, sd0)

    def step(c, carry):
        nxt = c + 1

        @pl.when(jnp.logical_and(nxt < NCHUNK, nxt % 2 == 0))
        def _():
            @pl.when(nxt >= 2)
            def _():
                wait_store(nxt - 2, eb0, so0)
            issue(nxt, eb0, sl0, is0, si0, id0, sd0)

        @pl.when(jnp.logical_and(nxt < NCHUNK, nxt % 2 == 1))
        def _():
            @pl.when(nxt >= 2)
            def _():
                wait_store(nxt - 2, eb1, so1)
            issue(nxt, eb1, sl1, is1, si1, id1, sd1)

        @pl.when(c % 2 == 0)
        def _():
            drain(c, eb0, sl0, is0, si0, id0, sd0, so0)

        @pl.when(c % 2 == 1)
        def _():
            drain(c, eb1, sl1, is1, si1, id1, sd1, so1)

        return carry

    lax.fori_loop(0, NCHUNK, step, 0)
    wait_store(NCHUNK - 2, eb1 if (NCHUNK - 2) % 2 else eb0,
               so1 if (NCHUNK - 2) % 2 else so0)
    wait_store(NCHUNK - 1, eb1 if (NCHUNK - 1) % 2 else eb0,
               so1 if (NCHUNK - 1) % 2 else so0)
    plsc.subcore_barrier()

    def dump(k, carry):
        rows = pl.ds(sid * ROWS_PER_TILE + k * CHUNK, CHUNK)
        pltpu.sync_copy(accum.at[rows], eb0)
        pltpu.sync_copy(eb0, acc_hbm.at[cid, rows])
        return carry

    lax.fori_loop(0, ROWS_PER_TILE // CHUNK, dump, 0)


@functools.cache
def _sc_kernels():
    mesh = plsc.VectorSubcoreMesh(
        core_axis_name="c", subcore_axis_name="s",
        num_cores=NC, num_subcores=NS)
    gather = pl.kernel(
        _sc_gather_body,
        out_type=jax.ShapeDtypeStruct((N_EDGES, HID), jnp.float32),
        mesh=mesh,
        scratch_types=[
            pltpu.VMEM((NCHUNK, CHUNK), jnp.int32),
            pltpu.VMEM((NCHUNK, CHUNK), jnp.int32),
            pltpu.VMEM((CHUNK, HID), jnp.float32),
            pltpu.VMEM((CHUNK, HID), jnp.float32),
            pltpu.VMEM((CHUNK, HID), jnp.float32),
            pltpu.VMEM((CHUNK, HID), jnp.float32),
            pltpu.SemaphoreType.DMA,
            pltpu.SemaphoreType.DMA,
            pltpu.SemaphoreType.DMA,
            pltpu.SemaphoreType.DMA,
            pltpu.SemaphoreType.DMA,
            pltpu.SemaphoreType.DMA,
        ],
    )
    scatter = pl.kernel(
        _sc_scatter_body,
        out_type=(
            jax.ShapeDtypeStruct((NC, NPAD, HID), jnp.float32),
            jax.ShapeDtypeStruct((N_EDGES, HID), jnp.float32),
        ),
        mesh=mesh,
        scratch_types=[
            pltpu.VMEM((CHUNK,), jnp.int32),
            pltpu.VMEM((CHUNK,), jnp.int32),
            pltpu.VMEM((CHUNK,), jnp.int32),
            pltpu.VMEM((CHUNK,), jnp.int32),
            pltpu.VMEM((CHUNK, HID), jnp.float32),
            pltpu.VMEM((CHUNK, HID), jnp.float32),
            pltpu.VMEM((2 * HID,), jnp.float32),
            pltpu.VMEM_SHARED((NPAD, HID), jnp.float32),
            pltpu.SemaphoreType.DMA,
            pltpu.SemaphoreType.DMA,
            pltpu.SemaphoreType.DMA,
            pltpu.SemaphoreType.DMA,
            pltpu.SemaphoreType.DMA,
            pltpu.SemaphoreType.DMA,
            pltpu.SemaphoreType.DMA,
            pltpu.SemaphoreType.DMA,
        ],
    )
    return gather, scatter


def _sc_gather(node_rep, src, dst):
    src3 = src.reshape(NW, NCHUNK, CHUNK)
    dst3 = dst.reshape(NW, NCHUNK, CHUNK)
    return _sc_kernels()[0](node_rep, src3, dst3)


def _sc_scatter(h2e, src, dst, scsh):
    src3 = src.reshape(NW, NCHUNK, CHUNK)
    dst3 = dst.reshape(NW, NCHUNK, CHUNK)
    return _sc_kernels()[1](h2e, src3, dst3, scsh)


BE = 2000
GE = N_EDGES // BE
BN = 1000
GN = N_NODES // BN


def _bn_affine(s_ref, q_ref, gam_ref, bet_ref, n):
    mu = s_ref[...] / n
    var = q_ref[...] / n - mu * mu
    scale = gam_ref[...] * lax.rsqrt(var + EPS)
    shift = bet_ref[...] - mu * scale
    return scale, shift


def _stats1_body(a_ref, b_ref, w1_ref, s_ref, q_ref):
    i = pl.program_id(0)
    x = jnp.concatenate([a_ref[...], b_ref[...]], axis=1)
    h = jnp.dot(x, w1_ref[...], preferred_element_type=jnp.float32)
    s = jnp.sum(h, axis=0, keepdims=True)
    q = jnp.sum(h * h, axis=0, keepdims=True)

    @pl.when(i == 0)
    def _():
        s_ref[...] = s
        q_ref[...] = q

    @pl.when(i > 0)
    def _():
        s_ref[...] += s
        q_ref[...] += q


def _stats1_node_body(a_ref, p0_ref, p1_ref, w1_ref, s_ref, q_ref):
    i = pl.program_id(0)
    x = jnp.concatenate([a_ref[...], p0_ref[...] + p1_ref[...]], axis=1)
    h = jnp.dot(x, w1_ref[...], preferred_element_type=jnp.float32)
    s = jnp.sum(h, axis=0, keepdims=True)
    q = jnp.sum(h * h, axis=0, keepdims=True)

    @pl.when(i == 0)
    def _():
        s_ref[...] = s
        q_ref[...] = q

    @pl.when(i > 0)
    def _():
        s_ref[...] += s
        q_ref[...] += q


def _mid_edge_body(n, ngrid, a_ref, b_ref, s1_ref, q1_ref, g1_ref, b1_ref,
                   w1_ref, w2_ref, g2_ref, b2_ref,
                   h2_ref, scale_ref, shift_ref):
    i = pl.program_id(0)
    scale, shift = _bn_affine(s1_ref, q1_ref, g1_ref, b1_ref, n)
    x = jnp.concatenate([a_ref[...], b_ref[...]], axis=1)
    h1 = jnp.dot(x, w1_ref[...], preferred_element_type=jnp.float32)
    a = jnp.maximum(h1 * scale + shift, 0.0)
    h2 = jnp.dot(a, w2_ref[...], preferred_element_type=jnp.float32)
    h2_ref[...] = h2
    s = jnp.sum(h2, axis=0, keepdims=True)
    q = jnp.sum(h2 * h2, axis=0, keepdims=True)

    @pl.when(i == 0)
    def _():
        scale_ref[...] = s
        shift_ref[...] = q

    @pl.when(i > 0)
    def _():
        scale_ref[...] += s
        shift_ref[...] += q

    @pl.when(i == ngrid - 1)
    def _():
        mu = scale_ref[...] / n
        var = shift_ref[...] / n - mu * mu
        sc2 = g2_ref[...] * lax.rsqrt(var + EPS)
        sh2 = b2_ref[...] - mu * sc2
        scale_ref[...] = sc2
        shift_ref[...] = sh2


def _mid_node_body(n, a_ref, p0_ref, p1_ref, s1_ref, q1_ref, g1_ref, b1_ref,
                   w1_ref, w2_ref, h2_ref, s_ref, q_ref):
    i = pl.program_id(0)
    scale, shift = _bn_affine(s1_ref, q1_ref, g1_ref, b1_ref, n)
    x = jnp.concatenate([a_ref[...], p0_ref[...] + p1_ref[...]], axis=1)
    h1 = jnp.dot(x, w1_ref[...], preferred_element_type=jnp.float32)
    a = jnp.maximum(h1 * scale + shift, 0.0)
    h2 = jnp.dot(a, w2_ref[...], preferred_element_type=jnp.float32)
    h2_ref[...] = h2
    s = jnp.sum(h2, axis=0, keepdims=True)
    q = jnp.sum(h2 * h2, axis=0, keepdims=True)

    @pl.when(i == 0)
    def _():
        s_ref[...] = s
        q_ref[...] = q

    @pl.when(i > 0)
    def _():
        s_ref[...] += s
        q_ref[...] += q


def _final_body(n, h2_ref, s2_ref, q2_ref, g2_ref, b2_ref, out_ref):
    scale, shift = _bn_affine(s2_ref, q2_ref, g2_ref, b2_ref, n)
    out_ref[...] = jnp.maximum(h2_ref[...] * scale + shift, 0.0)


def _row_spec(rows, cols):
    return pl.BlockSpec((rows, cols), lambda i: (i, 0))


def _rep_spec(rows, cols):
    return pl.BlockSpec((rows, cols), lambda i: (0, 0))


def kernel(node_rep, edge_rep, edge_index,
           W1e, g1e, b1e, W2e, g2e, b2e,
           W1n, g1n, b1n, W2n, g2n, b2n):
    f32 = jnp.float32
    src = edge_index[0]
    dst = edge_index[1]
    ne = float(N_EDGES)
    nn = float(N_NODES)

    g = _sc_gather(node_rep, src, dst)

    g1e_, b1e_, g2e_, b2e_ = (v.reshape(1, -1) for v in (g1e, b1e, g2e, b2e))
    g1n_, b1n_, g2n_, b2n_ = (v.reshape(1, -1) for v in (g1n, b1n, g2n, b2n))

    s1, q1 = pl.pallas_call(
        _stats1_body,
        grid=(GE,),
        in_specs=[
            _row_spec(BE, HID), _row_spec(BE, HID),
            _rep_spec(2 * HID, 2 * HID),
        ],
        out_specs=[_rep_spec(1, 2 * HID), _rep_spec(1, 2 * HID)],
        out_shape=[jax.ShapeDtypeStruct((1, 2 * HID), f32)] * 2,
    )(edge_rep, g, W1e)

    h2e, scale2, shift2 = pl.pallas_call(
        functools.partial(_mid_edge_body, ne, GE),
        grid=(GE,),
        in_specs=[
            _row_spec(BE, HID), _row_spec(BE, HID),
            _rep_spec(1, 2 * HID), _rep_spec(1, 2 * HID),
            _rep_spec(1, 2 * HID), _rep_spec(1, 2 * HID),
            _rep_spec(2 * HID, 2 * HID), _rep_spec(2 * HID, HID),
            _rep_spec(1, HID), _rep_spec(1, HID),
        ],
        out_specs=[_row_spec(BE, HID), _rep_spec(1, HID), _rep_spec(1, HID)],
        out_shape=[
            jax.ShapeDtypeStruct((N_EDGES, HID), f32),
            jax.ShapeDtypeStruct((1, HID), f32),
            jax.ShapeDtypeStruct((1, HID), f32),
        ],
    )(edge_rep, g, s1, q1, g1e_, b1e_, W1e, W2e, g2e_, b2e_)

    scsh = jnp.concatenate([scale2, shift2], axis=1).reshape(2 * HID)
    acc, edge_out = _sc_scatter(h2e, src, dst, scsh)
    p0, p1 = acc[0, :N_NODES], acc[1, :N_NODES]

    s1n, q1n = pl.pallas_call(
        _stats1_node_body,
        grid=(GN,),
        in_specs=[
            _row_spec(BN, HID), _row_spec(BN, HID), _row_spec(BN, HID),
            _rep_spec(2 * HID, 2 * HID),
        ],
        out_specs=[_rep_spec(1, 2 * HID), _rep_spec(1, 2 * HID)],
        out_shape=[jax.ShapeDtypeStruct((1, 2 * HID), f32)] * 2,
    )(node_rep, p0, p1, W1n)

    h2n, s2n, q2n = pl.pallas_call(
        functools.partial(_mid_node_body, nn),
        grid=(GN,),
        in_specs=[
            _row_spec(BN, HID), _row_spec(BN, HID), _row_spec(BN, HID),
            _rep_spec(1, 2 * HID), _rep_spec(1, 2 * HID),
            _rep_spec(1, 2 * HID), _rep_spec(1, 2 * HID),
            _rep_spec(2 * HID, 2 * HID), _rep_spec(2 * HID, HID),
        ],
        out_specs=[_row_spec(BN, HID), _rep_spec(1, HID), _rep_spec(1, HID)],
        out_shape=[
            jax.ShapeDtypeStruct((N_NODES, HID), f32),
            jax.ShapeDtypeStruct((1, HID), f32),
            jax.ShapeDtypeStruct((1, HID), f32),
        ],
    )(node_rep, p0, p1, s1n, q1n, g1n_, b1n_, W1n, W2n)

    node_out = pl.pallas_call(
        functools.partial(_final_body, nn),
        grid=(GN,),
        in_specs=[
            _row_spec(BN, HID),
            _rep_spec(1, HID), _rep_spec(1, HID),
            _rep_spec(1, HID), _rep_spec(1, HID),
        ],
        out_specs=_row_spec(BN, HID),
        out_shape=jax.ShapeDtypeStruct((N_NODES, HID), f32),
    )(h2n, s2n, q2n, g2n_, b2n_)

    return (node_out, edge_out)


# speedup vs baseline: 1.1442x; 1.1442x over previous
import functools

import jax
import jax.numpy as jnp
from jax import lax
from jax.experimental import pallas as pl
from jax.experimental.pallas import tpu as pltpu
from jax.experimental.pallas import tpu_sc as plsc

HID = 128
N_NODES = 10000
N_EDGES = 320000
EPS = 1e-5

NC, NS, LANES = 2, 16, 16
NW = NC * NS
EPW = N_EDGES // NW
CHUNK = 80
NCHUNK = EPW // CHUNK
NPAD = 10240
ROWS_PER_TILE = NPAD // NS

def _sc_gather_body(node_hbm, src_hbm, dst_hbm, g_hbm,
                    idx_s, idx_d, ba0, bb0, ba1, bb1,
                    sa0, sb0, sa1, sb1, ss0, ss1):
    wid = lax.axis_index("s") * NC + lax.axis_index("c")
    base = wid * EPW

    pltpu.sync_copy(src_hbm.at[wid], idx_s)
    pltpu.sync_copy(dst_hbm.at[wid], idx_d)

    def issue(ci, ba, bb, sa, sb):
        pltpu.async_copy(node_hbm.at[idx_s.at[ci]], ba, sa)
        pltpu.async_copy(node_hbm.at[idx_d.at[ci]], bb, sb)

    def drain(ci, ba, bb, sa, sb, ss):
        pltpu.make_async_copy(node_hbm.at[idx_s.at[ci]], ba, sa).wait()
        pltpu.make_async_copy(node_hbm.at[idx_d.at[ci]], bb, sb).wait()

        def row(r, c2):
            for j in range(HID // LANES):
                sl = pl.ds(j * LANES, LANES)
                plsc.addupdate(ba.at[r, sl], bb[r, sl])
            return c2

        lax.fori_loop(0, CHUNK, row, 0)
        pltpu.async_copy(ba, g_hbm.at[pl.ds(base + ci * CHUNK, CHUNK)], ss)

    def wait_store(ci, ba, ss):
        pltpu.make_async_copy(ba, g_hbm.at[pl.ds(base + ci * CHUNK, CHUNK)],
                              ss).wait()

    issue(0, ba0, bb0, sa0, sb0)

    def step(c, carry):
        nxt = c + 1

        @pl.when(jnp.logical_and(nxt < NCHUNK, nxt % 2 == 0))
        def _():
            @pl.when(nxt >= 2)
            def _():
                wait_store(nxt - 2, ba0, ss0)
            issue(nxt, ba0, bb0, sa0, sb0)

        @pl.when(jnp.logical_and(nxt < NCHUNK, nxt % 2 == 1))
        def _():
            @pl.when(nxt >= 2)
            def _():
                wait_store(nxt - 2, ba1, ss1)
            issue(nxt, ba1, bb1, sa1, sb1)

        @pl.when(c % 2 == 0)
        def _():
            drain(c, ba0, bb0, sa0, sb0, ss0)

        @pl.when(c % 2 == 1)
        def _():
            drain(c, ba1, bb1, sa1, sb1, ss1)

        return carry

    lax.fori_loop(0, NCHUNK, step, 0)
    wait_store(NCHUNK - 2, ba1 if (NCHUNK - 2) % 2 else ba0,
               ss1 if (NCHUNK - 2) % 2 else ss0)
    wait_store(NCHUNK - 1, ba1 if (NCHUNK - 1) % 2 else ba0,
               ss1 if (NCHUNK - 1) % 2 else ss0)


def _sc_scatter_body(h2_hbm, src_hbm, dst_hbm, scsh_hbm, acc_hbm, eout_hbm,
                     is0, is1, id0, id1, eb0, eb1, scsh, accum,
                     sl0, sl1, si0, si1, sd0, sd1, so0, so1):
    cid = lax.axis_index("c")
    sid = lax.axis_index("s")
    wid = sid * NC + cid
    base = wid * EPW
    nj = HID // LANES

    pltpu.sync_copy(scsh_hbm, scsh)

    def zrow(r, carry):
        for j in range(nj):
            eb0[r, pl.ds(j * LANES, LANES)] = jnp.zeros((LANES,), jnp.float32)
        return carry

    lax.fori_loop(0, CHUNK, zrow, 0)

    def zcp(k, carry):
        rows = pl.ds(sid * ROWS_PER_TILE + k * CHUNK, CHUNK)
        pltpu.sync_copy(eb0, accum.at[rows])
        return carry

    lax.fori_loop(0, ROWS_PER_TILE // CHUNK, zcp, 0)
    plsc.subcore_barrier()

    def issue(ci, eb, sl, isb, si, idb, sd):
        pltpu.async_copy(h2_hbm.at[pl.ds(base + ci * CHUNK, CHUNK)], eb, sl)
        pltpu.async_copy(src_hbm.at[wid, ci], isb, si)
        pltpu.async_copy(dst_hbm.at[wid, ci], idb, sd)

    def drain(ci, eb, sl, isb, si, idb, sd, so):
        pltpu.make_async_copy(
            h2_hbm.at[pl.ds(base + ci * CHUNK, CHUNK)], eb, sl).wait()
        pltpu.make_async_copy(src_hbm.at[wid, ci], isb, si).wait()
        pltpu.make_async_copy(dst_hbm.at[wid, ci], idb, sd).wait()
        scs = tuple(scsh[pl.ds(j * LANES, LANES)] for j in range(nj))
        shs = tuple(scsh[pl.ds(HID + j * LANES, LANES)] for j in range(nj))

        def row(r, carry):
            cs, ch = carry
            for j in range(nj):
                sl_ = pl.ds(j * LANES, LANES)
                eb[r, sl_] = jnp.maximum(eb[r, sl_] * cs[j] + ch[j], 0.0)
            return carry

        lax.fori_loop(0, CHUNK, row, (scs, shs))
        pltpu.async_copy(eb, eout_hbm.at[pl.ds(base + ci * CHUNK, CHUNK)], so)
        pltpu.sync_copy(eb, accum.at[isb], add=True)
        pltpu.sync_copy(eb, accum.at[idb], add=True)

    def wait_store(ci, eb, so):
        pltpu.make_async_copy(
            eb, eout_hbm.at[pl.ds(base + ci * CHUNK, CHUNK)], so).wait()

    issue(0, eb0, sl0, is0, si0, id0, sd0)

    def step(c, carry):
        nxt = c + 1

        @pl.when(jnp.logical_and(nxt < NCHUNK, nxt % 2 == 0))
        def _():
            @pl.when(nxt >= 2)
            def _():
                wait_store(nxt - 2, eb0, so0)
            issue(nxt, eb0, sl0, is0, si0, id0, sd0)

        @pl.when(jnp.logical_and(nxt < NCHUNK, nxt % 2 == 1))
        def _():
            @pl.when(nxt >= 2)
            def _():
                wait_store(nxt - 2, eb1, so1)
            issue(nxt, eb1, sl1, is1, si1, id1, sd1)

        @pl.when(c % 2 == 0)
        def _():
            drain(c, eb0, sl0, is0, si0, id0, sd0, so0)

        @pl.when(c % 2 == 1)
        def _():
            drain(c, eb1, sl1, is1, si1, id1, sd1, so1)

        return carry

    lax.fori_loop(0, NCHUNK, step, 0)
    wait_store(NCHUNK - 2, eb1 if (NCHUNK - 2) % 2 else eb0,
               so1 if (NCHUNK - 2) % 2 else so0)
    wait_store(NCHUNK - 1, eb1 if (NCHUNK - 1) % 2 else eb0,
               so1 if (NCHUNK - 1) % 2 else so0)
    plsc.subcore_barrier()

    def dump(k, carry):
        rows = pl.ds(sid * ROWS_PER_TILE + k * CHUNK, CHUNK)
        pltpu.sync_copy(accum.at[rows], eb0)
        pltpu.sync_copy(eb0, acc_hbm.at[cid, rows])
        return carry

    lax.fori_loop(0, ROWS_PER_TILE // CHUNK, dump, 0)


@functools.cache
def _sc_kernels():
    mesh = plsc.VectorSubcoreMesh(
        core_axis_name="c", subcore_axis_name="s",
        num_cores=NC, num_subcores=NS)
    gather = pl.kernel(
        _sc_gather_body,
        out_type=jax.ShapeDtypeStruct((N_EDGES, HID), jnp.float32),
        mesh=mesh,
        scratch_types=[
            pltpu.VMEM((NCHUNK, CHUNK), jnp.int32),
            pltpu.VMEM((NCHUNK, CHUNK), jnp.int32),
            pltpu.VMEM((CHUNK, HID), jnp.float32),
            pltpu.VMEM((CHUNK, HID), jnp.float32),
            pltpu.VMEM((CHUNK, HID), jnp.float32),
            pltpu.VMEM((CHUNK, HID), jnp.float32),
            pltpu.SemaphoreType.DMA,
            pltpu.SemaphoreType.DMA,
            pltpu.SemaphoreType.DMA,
            pltpu.SemaphoreType.DMA,
            pltpu.SemaphoreType.DMA,
            pltpu.SemaphoreType.DMA,
        ],
    )
    scatter = pl.kernel(
        _sc_scatter_body,
        out_type=(
            jax.ShapeDtypeStruct((NC, NPAD, HID), jnp.float32),
            jax.ShapeDtypeStruct((N_EDGES, HID), jnp.float32),
        ),
        mesh=mesh,
        scratch_types=[
            pltpu.VMEM((CHUNK,), jnp.int32),
            pltpu.VMEM((CHUNK,), jnp.int32),
            pltpu.VMEM((CHUNK,), jnp.int32),
            pltpu.VMEM((CHUNK,), jnp.int32),
            pltpu.VMEM((CHUNK, HID), jnp.float32),
            pltpu.VMEM((CHUNK, HID), jnp.float32),
            pltpu.VMEM((2 * HID,), jnp.float32),
            pltpu.VMEM_SHARED((NPAD, HID), jnp.float32),
            pltpu.SemaphoreType.DMA,
            pltpu.SemaphoreType.DMA,
            pltpu.SemaphoreType.DMA,
            pltpu.SemaphoreType.DMA,
            pltpu.SemaphoreType.DMA,
            pltpu.SemaphoreType.DMA,
            pltpu.SemaphoreType.DMA,
            pltpu.SemaphoreType.DMA,
        ],
    )
    return gather, scatter


def _sc_gather(node_rep, src, dst):
    src3 = src.reshape(NW, NCHUNK, CHUNK)
    dst3 = dst.reshape(NW, NCHUNK, CHUNK)
    return _sc_kernels()[0](node_rep, src3, dst3)


def _sc_scatter(h2e, src, dst, scsh):
    src3 = src.reshape(NW, NCHUNK, CHUNK)
    dst3 = dst.reshape(NW, NCHUNK, CHUNK)
    return _sc_kernels()[1](h2e, src3, dst3, scsh)


BE = 4000
GE = N_EDGES // BE
BN = 2000
GN = N_NODES // BN


def _bn_affine(s_ref, q_ref, gam_ref, bet_ref, n):
    mu = s_ref[...] / n
    var = q_ref[...] / n - mu * mu
    scale = gam_ref[...] * lax.rsqrt(var + EPS)
    shift = bet_ref[...] - mu * scale
    return scale, shift


def _stats1_body(a_ref, b_ref, w1_ref, s_ref, q_ref):
    i = pl.program_id(0)
    x = jnp.concatenate([a_ref[...], b_ref[...]], axis=1)
    h = jnp.dot(x, w1_ref[...], preferred_element_type=jnp.float32)
    s = jnp.sum(h, axis=0, keepdims=True)
    q = jnp.sum(h * h, axis=0, keepdims=True)

    @pl.when(i == 0)
    def _():
        s_ref[...] = s
        q_ref[...] = q

    @pl.when(i > 0)
    def _():
        s_ref[...] += s
        q_ref[...] += q


def _stats1_node_body(a_ref, p0_ref, p1_ref, w1_ref, s_ref, q_ref):
    i = pl.program_id(0)
    x = jnp.concatenate([a_ref[...], p0_ref[...] + p1_ref[...]], axis=1)
    h = jnp.dot(x, w1_ref[...], preferred_element_type=jnp.float32)
    s = jnp.sum(h, axis=0, keepdims=True)
    q = jnp.sum(h * h, axis=0, keepdims=True)

    @pl.when(i == 0)
    def _():
        s_ref[...] = s
        q_ref[...] = q

    @pl.when(i > 0)
    def _():
        s_ref[...] += s
        q_ref[...] += q


def _mid_edge_body(n, ngrid, a_ref, b_ref, s1_ref, q1_ref, g1_ref, b1_ref,
                   w1_ref, w2_ref, g2_ref, b2_ref,
                   h2_ref, scale_ref, shift_ref):
    i = pl.program_id(0)
    scale, shift = _bn_affine(s1_ref, q1_ref, g1_ref, b1_ref, n)
    x = jnp.concatenate([a_ref[...], b_ref[...]], axis=1)
    h1 = jnp.dot(x, w1_ref[...], preferred_element_type=jnp.float32)
    a = jnp.maximum(h1 * scale + shift, 0.0)
    h2 = jnp.dot(a, w2_ref[...], preferred_element_type=jnp.float32)
    h2_ref[...] = h2
    s = jnp.sum(h2, axis=0, keepdims=True)
    q = jnp.sum(h2 * h2, axis=0, keepdims=True)

    @pl.when(i == 0)
    def _():
        scale_ref[...] = s
        shift_ref[...] = q

    @pl.when(i > 0)
    def _():
        scale_ref[...] += s
        shift_ref[...] += q

    @pl.when(i == ngrid - 1)
    def _():
        mu = scale_ref[...] / n
        var = shift_ref[...] / n - mu * mu
        sc2 = g2_ref[...] * lax.rsqrt(var + EPS)
        sh2 = b2_ref[...] - mu * sc2
        scale_ref[...] = sc2
        shift_ref[...] = sh2


def _mid_node_body(n, a_ref, p0_ref, p1_ref, s1_ref, q1_ref, g1_ref, b1_ref,
                   w1_ref, w2_ref, h2_ref, s_ref, q_ref):
    i = pl.program_id(0)
    scale, shift = _bn_affine(s1_ref, q1_ref, g1_ref, b1_ref, n)
    x = jnp.concatenate([a_ref[...], p0_ref[...] + p1_ref[...]], axis=1)
    h1 = jnp.dot(x, w1_ref[...], preferred_element_type=jnp.float32)
    a = jnp.maximum(h1 * scale + shift, 0.0)
    h2 = jnp.dot(a, w2_ref[...], preferred_element_type=jnp.float32)
    h2_ref[...] = h2
    s = jnp.sum(h2, axis=0, keepdims=True)
    q = jnp.sum(h2 * h2, axis=0, keepdims=True)

    @pl.when(i == 0)
    def _():
        s_ref[...] = s
        q_ref[...] = q

    @pl.when(i > 0)
    def _():
        s_ref[...] += s
        q_ref[...] += q


def _final_body(n, h2_ref, s2_ref, q2_ref, g2_ref, b2_ref, out_ref):
    scale, shift = _bn_affine(s2_ref, q2_ref, g2_ref, b2_ref, n)
    out_ref[...] = jnp.maximum(h2_ref[...] * scale + shift, 0.0)


def _row_spec(rows, cols):
    return pl.BlockSpec((rows, cols), lambda i: (i, 0))


def _rep_spec(rows, cols):
    return pl.BlockSpec((rows, cols), lambda i: (0, 0))


def kernel(node_rep, edge_rep, edge_index,
           W1e, g1e, b1e, W2e, g2e, b2e,
           W1n, g1n, b1n, W2n, g2n, b2n):
    f32 = jnp.float32
    src = edge_index[0]
    dst = edge_index[1]
    ne = float(N_EDGES)
    nn = float(N_NODES)

    g = _sc_gather(node_rep, src, dst)

    g1e_, b1e_, g2e_, b2e_ = (v.reshape(1, -1) for v in (g1e, b1e, g2e, b2e))
    g1n_, b1n_, g2n_, b2n_ = (v.reshape(1, -1) for v in (g1n, b1n, g2n, b2n))

    s1, q1 = pl.pallas_call(
        _stats1_body,
        grid=(GE,),
        in_specs=[
            _row_spec(BE, HID), _row_spec(BE, HID),
            _rep_spec(2 * HID, 2 * HID),
        ],
        out_specs=[_rep_spec(1, 2 * HID), _rep_spec(1, 2 * HID)],
        out_shape=[jax.ShapeDtypeStruct((1, 2 * HID), f32)] * 2,
    )(edge_rep, g, W1e)

    h2e, scale2, shift2 = pl.pallas_call(
        functools.partial(_mid_edge_body, ne, GE),
        grid=(GE,),
        in_specs=[
            _row_spec(BE, HID), _row_spec(BE, HID),
            _rep_spec(1, 2 * HID), _rep_spec(1, 2 * HID),
            _rep_spec(1, 2 * HID), _rep_spec(1, 2 * HID),
            _rep_spec(2 * HID, 2 * HID), _rep_spec(2 * HID, HID),
            _rep_spec(1, HID), _rep_spec(1, HID),
        ],
        out_specs=[_row_spec(BE, HID), _rep_spec(1, HID), _rep_spec(1, HID)],
        out_shape=[
            jax.ShapeDtypeStruct((N_EDGES, HID), f32),
            jax.ShapeDtypeStruct((1, HID), f32),
            jax.ShapeDtypeStruct((1, HID), f32),
        ],
    )(edge_rep, g, s1, q1, g1e_, b1e_, W1e, W2e, g2e_, b2e_)

    scsh = jnp.concatenate([scale2, shift2], axis=1).reshape(2 * HID)
    acc, edge_out = _sc_scatter(h2e, src, dst, scsh)
    p0, p1 = acc[0, :N_NODES], acc[1, :N_NODES]

    s1n, q1n = pl.pallas_call(
        _stats1_node_body,
        grid=(GN,),
        in_specs=[
            _row_spec(BN, HID), _row_spec(BN, HID), _row_spec(BN, HID),
            _rep_spec(2 * HID, 2 * HID),
        ],
        out_specs=[_rep_spec(1, 2 * HID), _rep_spec(1, 2 * HID)],
        out_shape=[jax.ShapeDtypeStruct((1, 2 * HID), f32)] * 2,
    )(node_rep, p0, p1, W1n)

    h2n, s2n, q2n = pl.pallas_call(
        functools.partial(_mid_node_body, nn),
        grid=(GN,),
        in_specs=[
            _row_spec(BN, HID), _row_spec(BN, HID), _row_spec(BN, HID),
            _rep_spec(1, 2 * HID), _rep_spec(1, 2 * HID),
            _rep_spec(1, 2 * HID), _rep_spec(1, 2 * HID),
            _rep_spec(2 * HID, 2 * HID), _rep_spec(2 * HID, HID),
        ],
        out_specs=[_row_spec(BN, HID), _rep_spec(1, HID), _rep_spec(1, HID)],
        out_shape=[
            jax.ShapeDtypeStruct((N_NODES, HID), f32),
            jax.ShapeDtypeStruct((1, HID), f32),
            jax.ShapeDtypeStruct((1, HID), f32),
        ],
    )(node_rep, p0, p1, s1n, q1n, g1n_, b1n_, W1n, W2n)

    node_out = pl.pallas_call(
        functools.partial(_final_body, nn),
        grid=(GN,),
        in_specs=[
            _row_spec(BN, HID),
            _rep_spec(1, HID), _rep_spec(1, HID),
            _rep_spec(1, HID), _rep_spec(1, HID),
        ],
        out_specs=_row_spec(BN, HID),
        out_shape=jax.ShapeDtypeStruct((N_NODES, HID), f32),
    )(h2n, s2n, q2n, g2n_, b2n_)

    return (node_out, edge_out)


# speedup vs baseline: 1.2021x; 1.0506x over previous
import functools

import jax
import jax.numpy as jnp
from jax import lax
from jax.experimental import pallas as pl
from jax.experimental.pallas import tpu as pltpu
from jax.experimental.pallas import tpu_sc as plsc

HID = 128
N_NODES = 10000
N_EDGES = 320000
EPS = 1e-5

NC, NS, LANES = 2, 16, 16
NW = NC * NS
EPW = N_EDGES // NW
CHUNK = 80
NCHUNK = EPW // CHUNK
NPAD = 10240
ROWS_PER_TILE = NPAD // NS

def _sc_gather_body(node_hbm, src_hbm, dst_hbm, g_hbm,
                    idx_s, idx_d, ba0, bb0, ba1, bb1,
                    sa0, sb0, sa1, sb1, ss0, ss1):
    wid = lax.axis_index("s") * NC + lax.axis_index("c")
    base = wid * EPW

    pltpu.sync_copy(src_hbm.at[wid], idx_s)
    pltpu.sync_copy(dst_hbm.at[wid], idx_d)

    def issue(ci, ba, bb, sa, sb):
        pltpu.async_copy(node_hbm.at[idx_s.at[ci]], ba, sa)
        pltpu.async_copy(node_hbm.at[idx_d.at[ci]], bb, sb)

    def drain(ci, ba, bb, sa, sb, ss):
        pltpu.make_async_copy(node_hbm.at[idx_s.at[ci]], ba, sa).wait()
        pltpu.make_async_copy(node_hbm.at[idx_d.at[ci]], bb, sb).wait()

        def row(r, c2):
            for j in range(HID // LANES):
                sl = pl.ds(j * LANES, LANES)
                plsc.addupdate(ba.at[r, sl], bb[r, sl])
            return c2

        lax.fori_loop(0, CHUNK, row, 0)
        pltpu.async_copy(ba, g_hbm.at[pl.ds(base + ci * CHUNK, CHUNK)], ss)

    def wait_store(ci, ba, ss):
        pltpu.make_async_copy(ba, g_hbm.at[pl.ds(base + ci * CHUNK, CHUNK)],
                              ss).wait()

    issue(0, ba0, bb0, sa0, sb0)

    def step(c, carry):
        nxt = c + 1

        @pl.when(jnp.logical_and(nxt < NCHUNK, nxt % 2 == 0))
        def _():
            @pl.when(nxt >= 2)
            def _():
                wait_store(nxt - 2, ba0, ss0)
            issue(nxt, ba0, bb0, sa0, sb0)

        @pl.when(jnp.logical_and(nxt < NCHUNK, nxt % 2 == 1))
        def _():
            @pl.when(nxt >= 2)
            def _():
                wait_store(nxt - 2, ba1, ss1)
            issue(nxt, ba1, bb1, sa1, sb1)

        @pl.when(c % 2 == 0)
        def _():
            drain(c, ba0, bb0, sa0, sb0, ss0)

        @pl.when(c % 2 == 1)
        def _():
            drain(c, ba1, bb1, sa1, sb1, ss1)

        return carry

    lax.fori_loop(0, NCHUNK, step, 0)
    wait_store(NCHUNK - 2, ba1 if (NCHUNK - 2) % 2 else ba0,
               ss1 if (NCHUNK - 2) % 2 else ss0)
    wait_store(NCHUNK - 1, ba1 if (NCHUNK - 1) % 2 else ba0,
               ss1 if (NCHUNK - 1) % 2 else ss0)


def _sc_scatter_body(h2_hbm, src_hbm, dst_hbm, scsh_hbm, acc_hbm, eout_hbm,
                     is0, is1, id0, id1, eb0, eb1, scsh, accum,
                     sl0, sl1, si0, si1, sd0, sd1, so0, so1):
    cid = lax.axis_index("c")
    sid = lax.axis_index("s")
    wid = sid * NC + cid
    base = wid * EPW
    nj = HID // LANES

    pltpu.sync_copy(scsh_hbm, scsh)

    def zrow(r, carry):
        for j in range(nj):
            eb0[r, pl.ds(j * LANES, LANES)] = jnp.zeros((LANES,), jnp.float32)
        return carry

    lax.fori_loop(0, CHUNK, zrow, 0)

    def zcp(k, carry):
        rows = pl.ds(sid * ROWS_PER_TILE + k * CHUNK, CHUNK)
        pltpu.sync_copy(eb0, accum.at[rows])
        return carry

    lax.fori_loop(0, ROWS_PER_TILE // CHUNK, zcp, 0)
    plsc.subcore_barrier()

    def issue(ci, eb, sl, isb, si, idb, sd):
        pltpu.async_copy(h2_hbm.at[pl.ds(base + ci * CHUNK, CHUNK)], eb, sl)
        pltpu.async_copy(src_hbm.at[wid, ci], isb, si)
        pltpu.async_copy(dst_hbm.at[wid, ci], idb, sd)

    def drain(ci, eb, sl, isb, si, idb, sd, so):
        pltpu.make_async_copy(
            h2_hbm.at[pl.ds(base + ci * CHUNK, CHUNK)], eb, sl).wait()
        pltpu.make_async_copy(src_hbm.at[wid, ci], isb, si).wait()
        pltpu.make_async_copy(dst_hbm.at[wid, ci], idb, sd).wait()
        scs = tuple(scsh[pl.ds(j * LANES, LANES)] for j in range(nj))
        shs = tuple(scsh[pl.ds(HID + j * LANES, LANES)] for j in range(nj))

        def row(r, carry):
            cs, ch = carry
            for j in range(nj):
                sl_ = pl.ds(j * LANES, LANES)
                eb[r, sl_] = jnp.maximum(eb[r, sl_] * cs[j] + ch[j], 0.0)
            return carry

        lax.fori_loop(0, CHUNK, row, (scs, shs))
        pltpu.async_copy(eb, eout_hbm.at[pl.ds(base + ci * CHUNK, CHUNK)], so)
        pltpu.sync_copy(eb, accum.at[isb], add=True)
        pltpu.sync_copy(eb, accum.at[idb], add=True)

    def wait_store(ci, eb, so):
        pltpu.make_async_copy(
            eb, eout_hbm.at[pl.ds(base + ci * CHUNK, CHUNK)], so).wait()

    issue(0, eb0, sl0, is0, si0, id0, sd0)

    def step(c, carry):
        nxt = c + 1

        @pl.when(jnp.logical_and(nxt < NCHUNK, nxt % 2 == 0))
        def _():
            @pl.when(nxt >= 2)
            def _():
                wait_store(nxt - 2, eb0, so0)
            issue(nxt, eb0, sl0, is0, si0, id0, sd0)

        @pl.when(jnp.logical_and(nxt < NCHUNK, nxt % 2 == 1))
        def _():
            @pl.when(nxt >= 2)
            def _():
                wait_store(nxt - 2, eb1, so1)
            issue(nxt, eb1, sl1, is1, si1, id1, sd1)

        @pl.when(c % 2 == 0)
        def _():
            drain(c, eb0, sl0, is0, si0, id0, sd0, so0)

        @pl.when(c % 2 == 1)
        def _():
            drain(c, eb1, sl1, is1, si1, id1, sd1, so1)

        return carry

    lax.fori_loop(0, NCHUNK, step, 0)
    wait_store(NCHUNK - 2, eb1 if (NCHUNK - 2) % 2 else eb0,
               so1 if (NCHUNK - 2) % 2 else so0)
    wait_store(NCHUNK - 1, eb1 if (NCHUNK - 1) % 2 else eb0,
               so1 if (NCHUNK - 1) % 2 else so0)
    plsc.subcore_barrier()

    def dump(k, carry):
        rows = pl.ds(sid * ROWS_PER_TILE + k * CHUNK, CHUNK)
        pltpu.sync_copy(accum.at[rows], eb0)
        pltpu.sync_copy(eb0, acc_hbm.at[cid, rows])
        return carry

    lax.fori_loop(0, ROWS_PER_TILE // CHUNK, dump, 0)


@functools.cache
def _sc_kernels():
    mesh = plsc.VectorSubcoreMesh(
        core_axis_name="c", subcore_axis_name="s",
        num_cores=NC, num_subcores=NS)
    gather = pl.kernel(
        _sc_gather_body,
        out_type=jax.ShapeDtypeStruct((N_EDGES, HID), jnp.float32),
        mesh=mesh,
        scratch_types=[
            pltpu.VMEM((NCHUNK, CHUNK), jnp.int32),
            pltpu.VMEM((NCHUNK, CHUNK), jnp.int32),
            pltpu.VMEM((CHUNK, HID), jnp.float32),
            pltpu.VMEM((CHUNK, HID), jnp.float32),
            pltpu.VMEM((CHUNK, HID), jnp.float32),
            pltpu.VMEM((CHUNK, HID), jnp.float32),
            pltpu.SemaphoreType.DMA,
            pltpu.SemaphoreType.DMA,
            pltpu.SemaphoreType.DMA,
            pltpu.SemaphoreType.DMA,
            pltpu.SemaphoreType.DMA,
            pltpu.SemaphoreType.DMA,
        ],
    )
    scatter = pl.kernel(
        _sc_scatter_body,
        out_type=(
            jax.ShapeDtypeStruct((NC, NPAD, HID), jnp.float32),
            jax.ShapeDtypeStruct((N_EDGES, HID), jnp.float32),
        ),
        mesh=mesh,
        scratch_types=[
            pltpu.VMEM((CHUNK,), jnp.int32),
            pltpu.VMEM((CHUNK,), jnp.int32),
            pltpu.VMEM((CHUNK,), jnp.int32),
            pltpu.VMEM((CHUNK,), jnp.int32),
            pltpu.VMEM((CHUNK, HID), jnp.float32),
            pltpu.VMEM((CHUNK, HID), jnp.float32),
            pltpu.VMEM((2 * HID,), jnp.float32),
            pltpu.VMEM_SHARED((NPAD, HID), jnp.float32),
            pltpu.SemaphoreType.DMA,
            pltpu.SemaphoreType.DMA,
            pltpu.SemaphoreType.DMA,
            pltpu.SemaphoreType.DMA,
            pltpu.SemaphoreType.DMA,
            pltpu.SemaphoreType.DMA,
            pltpu.SemaphoreType.DMA,
            pltpu.SemaphoreType.DMA,
        ],
    )
    return gather, scatter


def _sc_gather(node_rep, src, dst):
    src3 = src.reshape(NW, NCHUNK, CHUNK)
    dst3 = dst.reshape(NW, NCHUNK, CHUNK)
    return _sc_kernels()[0](node_rep, src3, dst3)


def _sc_scatter(h2e, src, dst, scsh):
    src3 = src.reshape(NW, NCHUNK, CHUNK)
    dst3 = dst.reshape(NW, NCHUNK, CHUNK)
    return _sc_kernels()[1](h2e, src3, dst3, scsh)


BE = 8000
GE = N_EDGES // BE
BN = 2000
GN = N_NODES // BN


def _bn_affine(s_ref, q_ref, gam_ref, bet_ref, n):
    mu = s_ref[...] / n
    var = q_ref[...] / n - mu * mu
    scale = gam_ref[...] * lax.rsqrt(var + EPS)
    shift = bet_ref[...] - mu * scale
    return scale, shift


def _stats1_body(a_ref, b_ref, w1_ref, s_ref, q_ref):
    i = pl.program_id(0)
    x = jnp.concatenate([a_ref[...], b_ref[...]], axis=1)
    h = jnp.dot(x, w1_ref[...], preferred_element_type=jnp.float32)
    s = jnp.sum(h, axis=0, keepdims=True)
    q = jnp.sum(h * h, axis=0, keepdims=True)

    @pl.when(i == 0)
    def _():
        s_ref[...] = s
        q_ref[...] = q

    @pl.when(i > 0)
    def _():
        s_ref[...] += s
        q_ref[...] += q


def _stats1_node_body(a_ref, p0_ref, p1_ref, w1_ref, s_ref, q_ref):
    i = pl.program_id(0)
    x = jnp.concatenate([a_ref[...], p0_ref[...] + p1_ref[...]], axis=1)
    h = jnp.dot(x, w1_ref[...], preferred_element_type=jnp.float32)
    s = jnp.sum(h, axis=0, keepdims=True)
    q = jnp.sum(h * h, axis=0, keepdims=True)

    @pl.when(i == 0)
    def _():
        s_ref[...] = s
        q_ref[...] = q

    @pl.when(i > 0)
    def _():
        s_ref[...] += s
        q_ref[...] += q


def _mid_edge_body(n, ngrid, a_ref, b_ref, s1_ref, q1_ref, g1_ref, b1_ref,
                   w1_ref, w2_ref, g2_ref, b2_ref,
                   h2_ref, scale_ref, shift_ref):
    i = pl.program_id(0)
    scale, shift = _bn_affine(s1_ref, q1_ref, g1_ref, b1_ref, n)
    x = jnp.concatenate([a_ref[...], b_ref[...]], axis=1)
    h1 = jnp.dot(x, w1_ref[...], preferred_element_type=jnp.float32)
    a = jnp.maximum(h1 * scale + shift, 0.0)
    h2 = jnp.dot(a, w2_ref[...], preferred_element_type=jnp.float32)
    h2_ref[...] = h2
    s = jnp.sum(h2, axis=0, keepdims=True)
    q = jnp.sum(h2 * h2, axis=0, keepdims=True)

    @pl.when(i == 0)
    def _():
        scale_ref[...] = s
        shift_ref[...] = q

    @pl.when(i > 0)
    def _():
        scale_ref[...] += s
        shift_ref[...] += q

    @pl.when(i == ngrid - 1)
    def _():
        mu = scale_ref[...] / n
        var = shift_ref[...] / n - mu * mu
        sc2 = g2_ref[...] * lax.rsqrt(var + EPS)
        sh2 = b2_ref[...] - mu * sc2
        scale_ref[...] = sc2
        shift_ref[...] = sh2


def _mid_node_body(n, a_ref, p0_ref, p1_ref, s1_ref, q1_ref, g1_ref, b1_ref,
                   w1_ref, w2_ref, h2_ref, s_ref, q_ref):
    i = pl.program_id(0)
    scale, shift = _bn_affine(s1_ref, q1_ref, g1_ref, b1_ref, n)
    x = jnp.concatenate([a_ref[...], p0_ref[...] + p1_ref[...]], axis=1)
    h1 = jnp.dot(x, w1_ref[...], preferred_element_type=jnp.float32)
    a = jnp.maximum(h1 * scale + shift, 0.0)
    h2 = jnp.dot(a, w2_ref[...], preferred_element_type=jnp.float32)
    h2_ref[...] = h2
    s = jnp.sum(h2, axis=0, keepdims=True)
    q = jnp.sum(h2 * h2, axis=0, keepdims=True)

    @pl.when(i == 0)
    def _():
        s_ref[...] = s
        q_ref[...] = q

    @pl.when(i > 0)
    def _():
        s_ref[...] += s
        q_ref[...] += q


def _final_body(n, h2_ref, s2_ref, q2_ref, g2_ref, b2_ref, out_ref):
    scale, shift = _bn_affine(s2_ref, q2_ref, g2_ref, b2_ref, n)
    out_ref[...] = jnp.maximum(h2_ref[...] * scale + shift, 0.0)


def _row_spec(rows, cols):
    return pl.BlockSpec((rows, cols), lambda i: (i, 0))


def _rep_spec(rows, cols):
    return pl.BlockSpec((rows, cols), lambda i: (0, 0))


def kernel(node_rep, edge_rep, edge_index,
           W1e, g1e, b1e, W2e, g2e, b2e,
           W1n, g1n, b1n, W2n, g2n, b2n):
    f32 = jnp.float32
    src = edge_index[0]
    dst = edge_index[1]
    ne = float(N_EDGES)
    nn = float(N_NODES)

    g = _sc_gather(node_rep, src, dst)

    g1e_, b1e_, g2e_, b2e_ = (v.reshape(1, -1) for v in (g1e, b1e, g2e, b2e))
    g1n_, b1n_, g2n_, b2n_ = (v.reshape(1, -1) for v in (g1n, b1n, g2n, b2n))

    s1, q1 = pl.pallas_call(
        _stats1_body,
        grid=(GE,),
        in_specs=[
            _row_spec(BE, HID), _row_spec(BE, HID),
            _rep_spec(2 * HID, 2 * HID),
        ],
        out_specs=[_rep_spec(1, 2 * HID), _rep_spec(1, 2 * HID)],
        out_shape=[jax.ShapeDtypeStruct((1, 2 * HID), f32)] * 2,
    )(edge_rep, g, W1e)

    h2e, scale2, shift2 = pl.pallas_call(
        functools.partial(_mid_edge_body, ne, GE),
        grid=(GE,),
        in_specs=[
            _row_spec(BE, HID), _row_spec(BE, HID),
            _rep_spec(1, 2 * HID), _rep_spec(1, 2 * HID),
            _rep_spec(1, 2 * HID), _rep_spec(1, 2 * HID),
            _rep_spec(2 * HID, 2 * HID), _rep_spec(2 * HID, HID),
            _rep_spec(1, HID), _rep_spec(1, HID),
        ],
        out_specs=[_row_spec(BE, HID), _rep_spec(1, HID), _rep_spec(1, HID)],
        out_shape=[
            jax.ShapeDtypeStruct((N_EDGES, HID), f32),
            jax.ShapeDtypeStruct((1, HID), f32),
            jax.ShapeDtypeStruct((1, HID), f32),
        ],
    )(edge_rep, g, s1, q1, g1e_, b1e_, W1e, W2e, g2e_, b2e_)

    scsh = jnp.concatenate([scale2, shift2], axis=1).reshape(2 * HID)
    acc, edge_out = _sc_scatter(h2e, src, dst, scsh)
    p0, p1 = acc[0, :N_NODES], acc[1, :N_NODES]

    s1n, q1n = pl.pallas_call(
        _stats1_node_body,
        grid=(GN,),
        in_specs=[
            _row_spec(BN, HID), _row_spec(BN, HID), _row_spec(BN, HID),
            _rep_spec(2 * HID, 2 * HID),
        ],
        out_specs=[_rep_spec(1, 2 * HID), _rep_spec(1, 2 * HID)],
        out_shape=[jax.ShapeDtypeStruct((1, 2 * HID), f32)] * 2,
    )(node_rep, p0, p1, W1n)

    h2n, s2n, q2n = pl.pallas_call(
        functools.partial(_mid_node_body, nn),
        grid=(GN,),
        in_specs=[
            _row_spec(BN, HID), _row_spec(BN, HID), _row_spec(BN, HID),
            _rep_spec(1, 2 * HID), _rep_spec(1, 2 * HID),
            _rep_spec(1, 2 * HID), _rep_spec(1, 2 * HID),
            _rep_spec(2 * HID, 2 * HID), _rep_spec(2 * HID, HID),
        ],
        out_specs=[_row_spec(BN, HID), _rep_spec(1, HID), _rep_spec(1, HID)],
        out_shape=[
            jax.ShapeDtypeStruct((N_NODES, HID), f32),
            jax.ShapeDtypeStruct((1, HID), f32),
            jax.ShapeDtypeStruct((1, HID), f32),
        ],
    )(node_rep, p0, p1, s1n, q1n, g1n_, b1n_, W1n, W2n)

    node_out = pl.pallas_call(
        functools.partial(_final_body, nn),
        grid=(GN,),
        in_specs=[
            _row_spec(BN, HID),
            _rep_spec(1, HID), _rep_spec(1, HID),
            _rep_spec(1, HID), _rep_spec(1, HID),
        ],
        out_specs=_row_spec(BN, HID),
        out_shape=jax.ShapeDtypeStruct((N_NODES, HID), f32),
    )(h2n, s2n, q2n, g2n_, b2n_)

    return (node_out, edge_out)


# speedup vs baseline: 1.2251x; 1.0192x over previous
import functools

import jax
import jax.numpy as jnp
from jax import lax
from jax.experimental import pallas as pl
from jax.experimental.pallas import tpu as pltpu
from jax.experimental.pallas import tpu_sc as plsc

HID = 128
N_NODES = 10000
N_EDGES = 320000
EPS = 1e-5

NC, NS, LANES = 2, 16, 16
NW = NC * NS
EPW = N_EDGES // NW
CHUNK = 80
NCHUNK = EPW // CHUNK
NPAD = 10240
ROWS_PER_TILE = NPAD // NS

def _sc_gather_body(node_hbm, src_hbm, dst_hbm, g_hbm,
                    idx_s, idx_d, ba0, bb0, ba1, bb1,
                    sa0, sb0, sa1, sb1, ss0, ss1):
    wid = lax.axis_index("s") * NC + lax.axis_index("c")
    base = wid * EPW

    pltpu.sync_copy(src_hbm.at[wid], idx_s)
    pltpu.sync_copy(dst_hbm.at[wid], idx_d)

    def issue(ci, ba, bb, sa, sb):
        pltpu.async_copy(node_hbm.at[idx_s.at[ci]], ba, sa)
        pltpu.async_copy(node_hbm.at[idx_d.at[ci]], bb, sb)

    def drain(ci, ba, bb, sa, sb, ss):
        pltpu.make_async_copy(node_hbm.at[idx_s.at[ci]], ba, sa).wait()
        pltpu.make_async_copy(node_hbm.at[idx_d.at[ci]], bb, sb).wait()

        def row(r, c2):
            for j in range(HID // LANES):
                sl = pl.ds(j * LANES, LANES)
                plsc.addupdate(ba.at[r, sl], bb[r, sl])
            return c2

        lax.fori_loop(0, CHUNK, row, 0)
        pltpu.async_copy(ba, g_hbm.at[pl.ds(base + ci * CHUNK, CHUNK)], ss)

    def wait_store(ci, ba, ss):
        pltpu.make_async_copy(ba, g_hbm.at[pl.ds(base + ci * CHUNK, CHUNK)],
                              ss).wait()

    issue(0, ba0, bb0, sa0, sb0)

    def step(c, carry):
        nxt = c + 1

        @pl.when(jnp.logical_and(nxt < NCHUNK, nxt % 2 == 0))
        def _():
            @pl.when(nxt >= 2)
            def _():
                wait_store(nxt - 2, ba0, ss0)
            issue(nxt, ba0, bb0, sa0, sb0)

        @pl.when(jnp.logical_and(nxt < NCHUNK, nxt % 2 == 1))
        def _():
            @pl.when(nxt >= 2)
            def _():
                wait_store(nxt - 2, ba1, ss1)
            issue(nxt, ba1, bb1, sa1, sb1)

        @pl.when(c % 2 == 0)
        def _():
            drain(c, ba0, bb0, sa0, sb0, ss0)

        @pl.when(c % 2 == 1)
        def _():
            drain(c, ba1, bb1, sa1, sb1, ss1)

        return carry

    lax.fori_loop(0, NCHUNK, step, 0)
    wait_store(NCHUNK - 2, ba1 if (NCHUNK - 2) % 2 else ba0,
               ss1 if (NCHUNK - 2) % 2 else ss0)
    wait_store(NCHUNK - 1, ba1 if (NCHUNK - 1) % 2 else ba0,
               ss1 if (NCHUNK - 1) % 2 else ss0)


def _sc_scatter_body(h2_hbm, src_hbm, dst_hbm, scsh_hbm, acc_hbm, eout_hbm,
                     is0, is1, id0, id1, eb0, eb1, scsh, accum,
                     sl0, sl1, si0, si1, sd0, sd1, so0, so1):
    cid = lax.axis_index("c")
    sid = lax.axis_index("s")
    wid = sid * NC + cid
    base = wid * EPW
    nj = HID // LANES

    pltpu.sync_copy(scsh_hbm, scsh)

    def zrow(r, carry):
        for j in range(nj):
            eb0[r, pl.ds(j * LANES, LANES)] = jnp.zeros((LANES,), jnp.float32)
        return carry

    lax.fori_loop(0, CHUNK, zrow, 0)

    def zcp(k, carry):
        rows = pl.ds(sid * ROWS_PER_TILE + k * CHUNK, CHUNK)
        pltpu.sync_copy(eb0, accum.at[rows])
        return carry

    lax.fori_loop(0, ROWS_PER_TILE // CHUNK, zcp, 0)
    plsc.subcore_barrier()

    def issue(ci, eb, sl, isb, si, idb, sd):
        pltpu.async_copy(h2_hbm.at[pl.ds(base + ci * CHUNK, CHUNK)], eb, sl)
        pltpu.async_copy(src_hbm.at[wid, ci], isb, si)
        pltpu.async_copy(dst_hbm.at[wid, ci], idb, sd)

    def drain(ci, eb, sl, isb, si, idb, sd, so):
        pltpu.make_async_copy(
            h2_hbm.at[pl.ds(base + ci * CHUNK, CHUNK)], eb, sl).wait()
        pltpu.make_async_copy(src_hbm.at[wid, ci], isb, si).wait()
        pltpu.make_async_copy(dst_hbm.at[wid, ci], idb, sd).wait()
        scs = tuple(scsh[pl.ds(j * LANES, LANES)] for j in range(nj))
        shs = tuple(scsh[pl.ds(HID + j * LANES, LANES)] for j in range(nj))

        def row(r, carry):
            cs, ch = carry
            for j in range(nj):
                sl_ = pl.ds(j * LANES, LANES)
                eb[r, sl_] = jnp.maximum(eb[r, sl_] * cs[j] + ch[j], 0.0)
            return carry

        lax.fori_loop(0, CHUNK, row, (scs, shs))
        pltpu.async_copy(eb, eout_hbm.at[pl.ds(base + ci * CHUNK, CHUNK)], so)
        pltpu.sync_copy(eb, accum.at[isb], add=True)
        pltpu.sync_copy(eb, accum.at[idb], add=True)

    def wait_store(ci, eb, so):
        pltpu.make_async_copy(
            eb, eout_hbm.at[pl.ds(base + ci * CHUNK, CHUNK)], so).wait()

    issue(0, eb0, sl0, is0, si0, id0, sd0)

    def step(c, carry):
        nxt = c + 1

        @pl.when(jnp.logical_and(nxt < NCHUNK, nxt % 2 == 0))
        def _():
            @pl.when(nxt >= 2)
            def _():
                wait_store(nxt - 2, eb0, so0)
            issue(nxt, eb0, sl0, is0, si0, id0, sd0)

        @pl.when(jnp.logical_and(nxt < NCHUNK, nxt % 2 == 1))
        def _():
            @pl.when(nxt >= 2)
            def _():
                wait_store(nxt - 2, eb1, so1)
            issue(nxt, eb1, sl1, is1, si1, id1, sd1)

        @pl.when(c % 2 == 0)
        def _():
            drain(c, eb0, sl0, is0, si0, id0, sd0, so0)

        @pl.when(c % 2 == 1)
        def _():
            drain(c, eb1, sl1, is1, si1, id1, sd1, so1)

        return carry

    lax.fori_loop(0, NCHUNK, step, 0)
    wait_store(NCHUNK - 2, eb1 if (NCHUNK - 2) % 2 else eb0,
               so1 if (NCHUNK - 2) % 2 else so0)
    wait_store(NCHUNK - 1, eb1 if (NCHUNK - 1) % 2 else eb0,
               so1 if (NCHUNK - 1) % 2 else so0)
    plsc.subcore_barrier()

    def dump(k, carry):
        rows = pl.ds(sid * ROWS_PER_TILE + k * CHUNK, CHUNK)
        pltpu.sync_copy(accum.at[rows], eb0)
        pltpu.sync_copy(eb0, acc_hbm.at[cid, rows])
        return carry

    lax.fori_loop(0, ROWS_PER_TILE // CHUNK, dump, 0)


@functools.cache
def _sc_kernels():
    mesh = plsc.VectorSubcoreMesh(
        core_axis_name="c", subcore_axis_name="s",
        num_cores=NC, num_subcores=NS)
    gather = pl.kernel(
        _sc_gather_body,
        out_type=jax.ShapeDtypeStruct((N_EDGES, HID), jnp.float32),
        mesh=mesh,
        scratch_types=[
            pltpu.VMEM((NCHUNK, CHUNK), jnp.int32),
            pltpu.VMEM((NCHUNK, CHUNK), jnp.int32),
            pltpu.VMEM((CHUNK, HID), jnp.float32),
            pltpu.VMEM((CHUNK, HID), jnp.float32),
            pltpu.VMEM((CHUNK, HID), jnp.float32),
            pltpu.VMEM((CHUNK, HID), jnp.float32),
            pltpu.SemaphoreType.DMA,
            pltpu.SemaphoreType.DMA,
            pltpu.SemaphoreType.DMA,
            pltpu.SemaphoreType.DMA,
            pltpu.SemaphoreType.DMA,
            pltpu.SemaphoreType.DMA,
        ],
    )
    scatter = pl.kernel(
        _sc_scatter_body,
        out_type=(
            jax.ShapeDtypeStruct((NC, NPAD, HID), jnp.float32),
            jax.ShapeDtypeStruct((N_EDGES, HID), jnp.float32),
        ),
        mesh=mesh,
        scratch_types=[
            pltpu.VMEM((CHUNK,), jnp.int32),
            pltpu.VMEM((CHUNK,), jnp.int32),
            pltpu.VMEM((CHUNK,), jnp.int32),
            pltpu.VMEM((CHUNK,), jnp.int32),
            pltpu.VMEM((CHUNK, HID), jnp.float32),
            pltpu.VMEM((CHUNK, HID), jnp.float32),
            pltpu.VMEM((2 * HID,), jnp.float32),
            pltpu.VMEM_SHARED((NPAD, HID), jnp.float32),
            pltpu.SemaphoreType.DMA,
            pltpu.SemaphoreType.DMA,
            pltpu.SemaphoreType.DMA,
            pltpu.SemaphoreType.DMA,
            pltpu.SemaphoreType.DMA,
            pltpu.SemaphoreType.DMA,
            pltpu.SemaphoreType.DMA,
            pltpu.SemaphoreType.DMA,
        ],
    )
    return gather, scatter


def _sc_gather(node_rep, src, dst):
    src3 = src.reshape(NW, NCHUNK, CHUNK)
    dst3 = dst.reshape(NW, NCHUNK, CHUNK)
    return _sc_kernels()[0](node_rep, src3, dst3)


def _sc_scatter(h2e, src, dst, scsh):
    src3 = src.reshape(NW, NCHUNK, CHUNK)
    dst3 = dst.reshape(NW, NCHUNK, CHUNK)
    return _sc_kernels()[1](h2e, src3, dst3, scsh)


BE = 16000
GE = N_EDGES // BE
BN = 2000
GN = N_NODES // BN


def _bn_affine(s_ref, q_ref, gam_ref, bet_ref, n):
    mu = s_ref[...] / n
    var = q_ref[...] / n - mu * mu
    scale = gam_ref[...] * lax.rsqrt(var + EPS)
    shift = bet_ref[...] - mu * scale
    return scale, shift


def _stats1_body(a_ref, b_ref, w1_ref, s_ref, q_ref):
    i = pl.program_id(0)
    x = jnp.concatenate([a_ref[...], b_ref[...]], axis=1)
    h = jnp.dot(x, w1_ref[...], preferred_element_type=jnp.float32)
    s = jnp.sum(h, axis=0, keepdims=True)
    q = jnp.sum(h * h, axis=0, keepdims=True)

    @pl.when(i == 0)
    def _():
        s_ref[...] = s
        q_ref[...] = q

    @pl.when(i > 0)
    def _():
        s_ref[...] += s
        q_ref[...] += q


def _stats1_node_body(a_ref, p0_ref, p1_ref, w1_ref, s_ref, q_ref):
    i = pl.program_id(0)
    x = jnp.concatenate([a_ref[...], p0_ref[...] + p1_ref[...]], axis=1)
    h = jnp.dot(x, w1_ref[...], preferred_element_type=jnp.float32)
    s = jnp.sum(h, axis=0, keepdims=True)
    q = jnp.sum(h * h, axis=0, keepdims=True)

    @pl.when(i == 0)
    def _():
        s_ref[...] = s
        q_ref[...] = q

    @pl.when(i > 0)
    def _():
        s_ref[...] += s
        q_ref[...] += q


def _mid_edge_body(n, ngrid, a_ref, b_ref, s1_ref, q1_ref, g1_ref, b1_ref,
                   w1_ref, w2_ref, g2_ref, b2_ref,
                   h2_ref, scale_ref, shift_ref):
    i = pl.program_id(0)
    scale, shift = _bn_affine(s1_ref, q1_ref, g1_ref, b1_ref, n)
    x = jnp.concatenate([a_ref[...], b_ref[...]], axis=1)
    h1 = jnp.dot(x, w1_ref[...], preferred_element_type=jnp.float32)
    a = jnp.maximum(h1 * scale + shift, 0.0)
    h2 = jnp.dot(a, w2_ref[...], preferred_element_type=jnp.float32)
    h2_ref[...] = h2
    s = jnp.sum(h2, axis=0, keepdims=True)
    q = jnp.sum(h2 * h2, axis=0, keepdims=True)

    @pl.when(i == 0)
    def _():
        scale_ref[...] = s
        shift_ref[...] = q

    @pl.when(i > 0)
    def _():
        scale_ref[...] += s
        shift_ref[...] += q

    @pl.when(i == ngrid - 1)
    def _():
        mu = scale_ref[...] / n
        var = shift_ref[...] / n - mu * mu
        sc2 = g2_ref[...] * lax.rsqrt(var + EPS)
        sh2 = b2_ref[...] - mu * sc2
        scale_ref[...] = sc2
        shift_ref[...] = sh2


def _mid_node_body(n, a_ref, p0_ref, p1_ref, s1_ref, q1_ref, g1_ref, b1_ref,
                   w1_ref, w2_ref, h2_ref, s_ref, q_ref):
    i = pl.program_id(0)
    scale, shift = _bn_affine(s1_ref, q1_ref, g1_ref, b1_ref, n)
    x = jnp.concatenate([a_ref[...], p0_ref[...] + p1_ref[...]], axis=1)
    h1 = jnp.dot(x, w1_ref[...], preferred_element_type=jnp.float32)
    a = jnp.maximum(h1 * scale + shift, 0.0)
    h2 = jnp.dot(a, w2_ref[...], preferred_element_type=jnp.float32)
    h2_ref[...] = h2
    s = jnp.sum(h2, axis=0, keepdims=True)
    q = jnp.sum(h2 * h2, axis=0, keepdims=True)

    @pl.when(i == 0)
    def _():
        s_ref[...] = s
        q_ref[...] = q

    @pl.when(i > 0)
    def _():
        s_ref[...] += s
        q_ref[...] += q


def _final_body(n, h2_ref, s2_ref, q2_ref, g2_ref, b2_ref, out_ref):
    scale, shift = _bn_affine(s2_ref, q2_ref, g2_ref, b2_ref, n)
    out_ref[...] = jnp.maximum(h2_ref[...] * scale + shift, 0.0)


def _row_spec(rows, cols):
    return pl.BlockSpec((rows, cols), lambda i: (i, 0))


def _rep_spec(rows, cols):
    return pl.BlockSpec((rows, cols), lambda i: (0, 0))


def kernel(node_rep, edge_rep, edge_index,
           W1e, g1e, b1e, W2e, g2e, b2e,
           W1n, g1n, b1n, W2n, g2n, b2n):
    f32 = jnp.float32
    src = edge_index[0]
    dst = edge_index[1]
    ne = float(N_EDGES)
    nn = float(N_NODES)

    g = _sc_gather(node_rep, src, dst)

    g1e_, b1e_, g2e_, b2e_ = (v.reshape(1, -1) for v in (g1e, b1e, g2e, b2e))
    g1n_, b1n_, g2n_, b2n_ = (v.reshape(1, -1) for v in (g1n, b1n, g2n, b2n))

    s1, q1 = pl.pallas_call(
        _stats1_body,
        grid=(GE,),
        in_specs=[
            _row_spec(BE, HID), _row_spec(BE, HID),
            _rep_spec(2 * HID, 2 * HID),
        ],
        out_specs=[_rep_spec(1, 2 * HID), _rep_spec(1, 2 * HID)],
        out_shape=[jax.ShapeDtypeStruct((1, 2 * HID), f32)] * 2,
    )(edge_rep, g, W1e)

    h2e, scale2, shift2 = pl.pallas_call(
        functools.partial(_mid_edge_body, ne, GE),
        grid=(GE,),
        in_specs=[
            _row_spec(BE, HID), _row_spec(BE, HID),
            _rep_spec(1, 2 * HID), _rep_spec(1, 2 * HID),
            _rep_spec(1, 2 * HID), _rep_spec(1, 2 * HID),
            _rep_spec(2 * HID, 2 * HID), _rep_spec(2 * HID, HID),
            _rep_spec(1, HID), _rep_spec(1, HID),
        ],
        out_specs=[_row_spec(BE, HID), _rep_spec(1, HID), _rep_spec(1, HID)],
        out_shape=[
            jax.ShapeDtypeStruct((N_EDGES, HID), f32),
            jax.ShapeDtypeStruct((1, HID), f32),
            jax.ShapeDtypeStruct((1, HID), f32),
        ],
    )(edge_rep, g, s1, q1, g1e_, b1e_, W1e, W2e, g2e_, b2e_)

    scsh = jnp.concatenate([scale2, shift2], axis=1).reshape(2 * HID)
    acc, edge_out = _sc_scatter(h2e, src, dst, scsh)
    p0, p1 = acc[0, :N_NODES], acc[1, :N_NODES]

    s1n, q1n = pl.pallas_call(
        _stats1_node_body,
        grid=(GN,),
        in_specs=[
            _row_spec(BN, HID), _row_spec(BN, HID), _row_spec(BN, HID),
            _rep_spec(2 * HID, 2 * HID),
        ],
        out_specs=[_rep_spec(1, 2 * HID), _rep_spec(1, 2 * HID)],
        out_shape=[jax.ShapeDtypeStruct((1, 2 * HID), f32)] * 2,
    )(node_rep, p0, p1, W1n)

    h2n, s2n, q2n = pl.pallas_call(
        functools.partial(_mid_node_body, nn),
        grid=(GN,),
        in_specs=[
            _row_spec(BN, HID), _row_spec(BN, HID), _row_spec(BN, HID),
            _rep_spec(1, 2 * HID), _rep_spec(1, 2 * HID),
            _rep_spec(1, 2 * HID), _rep_spec(1, 2 * HID),
            _rep_spec(2 * HID, 2 * HID), _rep_spec(2 * HID, HID),
        ],
        out_specs=[_row_spec(BN, HID), _rep_spec(1, HID), _rep_spec(1, HID)],
        out_shape=[
            jax.ShapeDtypeStruct((N_NODES, HID), f32),
            jax.ShapeDtypeStruct((1, HID), f32),
            jax.ShapeDtypeStruct((1, HID), f32),
        ],
    )(node_rep, p0, p1, s1n, q1n, g1n_, b1n_, W1n, W2n)

    node_out = pl.pallas_call(
        functools.partial(_final_body, nn),
        grid=(GN,),
        in_specs=[
            _row_spec(BN, HID),
            _rep_spec(1, HID), _rep_spec(1, HID),
            _rep_spec(1, HID), _rep_spec(1, HID),
        ],
        out_specs=_row_spec(BN, HID),
        out_shape=jax.ShapeDtypeStruct((N_NODES, HID), f32),
    )(h2n, s2n, q2n, g2n_, b2n_)

    return (node_out, edge_out)
